# Initial kernel scaffold; baseline (speedup 1.0000x reference)
import jax
import jax.numpy as jnp
from jax.experimental import pallas as pl


def kernel(nodes, to_neighs, u2e_weight, W1, b1, W2, b2, W3, b3):
    raise NotImplementedError("write your pallas kernel here")


# speedup vs baseline: 3.3256x; 3.3256x over previous
import functools

import jax
import jax.numpy as jnp
from jax import lax
from jax.experimental import pallas as pl
from jax.experimental.pallas import tpu as pltpu
from jax.experimental.pallas import tpu_sc as plsc

NUM_USERS = 100000
EMBED = 128
N_NODES = 10000
DEG = 32

NC, NS = 2, 16
NW = NC * NS

E_ROWS = N_NODES * DEG
E_PER_W = E_ROWS // NW
CHUNK = 400
N_CHUNKS = E_PER_W // CHUNK

U_PAD = 10240
U_PER_W = U_PAD // NW


def _sc_gather_body(neigh_hbm, node_hbm, table_hbm,
                    e_out_hbm, u_out_hbm,
                    idx_v, rows_v, idx_u, rows_u, sem):
    wid = lax.axis_index("s") * NC + lax.axis_index("c")
    ebase = wid * E_PER_W

    def chunk(i, _):
        base = ebase + i * CHUNK
        pltpu.sync_copy(neigh_hbm.at[pl.ds(base, CHUNK)], idx_v)
        pltpu.async_copy(table_hbm.at[idx_v], rows_v, sem).wait()
        pltpu.sync_copy(rows_v, e_out_hbm.at[pl.ds(base, CHUNK)])
        return 0

    lax.fori_loop(0, N_CHUNKS, chunk, 0)

    ubase = wid * U_PER_W
    pltpu.sync_copy(node_hbm.at[pl.ds(ubase, U_PER_W)], idx_u)
    pltpu.async_copy(table_hbm.at[idx_u], rows_u, sem).wait()
    pltpu.sync_copy(rows_u, u_out_hbm.at[pl.ds(ubase, U_PER_W)])


@jax.jit
def _sc_gather(neigh_flat, nodes_pad, table):
    mesh = plsc.VectorSubcoreMesh(core_axis_name="c", subcore_axis_name="s")
    return pl.kernel(
        _sc_gather_body,
        out_type=(
            jax.ShapeDtypeStruct((E_ROWS, EMBED), jnp.float32),
            jax.ShapeDtypeStruct((U_PAD, EMBED), jnp.float32),
        ),
        mesh=mesh,
        scratch_types=[
            pltpu.VMEM((CHUNK,), jnp.int32),
            pltpu.VMEM((CHUNK, EMBED), jnp.float32),
            pltpu.VMEM((U_PER_W,), jnp.int32),
            pltpu.VMEM((U_PER_W, EMBED), jnp.float32),
            pltpu.SemaphoreType.DMA,
        ],
    )(neigh_flat, nodes_pad, table)


T = 400


def _tc_mlp_body(e_ref, u_ref, w1a_ref, w1b_ref, b1_ref, w2_ref, b2_ref,
                 w3_ref, out_ref):
    e3 = e_ref[...]
    e2 = e3.reshape(T * DEG, EMBED)
    u1 = jnp.dot(u_ref[...], w1b_ref[...],
                 preferred_element_type=jnp.float32)
    x = jnp.dot(e2, w1a_ref[...], preferred_element_type=jnp.float32)
    x = x.reshape(T, DEG, EMBED) + u1[:, None, :] + b1_ref[...][0][None, None, :]
    x = jnp.maximum(x, 0.0)
    x2 = jnp.dot(x.reshape(T * DEG, EMBED), w2_ref[...],
                 preferred_element_type=jnp.float32)
    x2 = jnp.maximum(x2 + b2_ref[...][0][None, :], 0.0)
    logits = jnp.sum(x2.reshape(T, DEG, EMBED) * w3_ref[...][0][None, None, :],
                     axis=-1)
    m = jnp.max(logits, axis=1, keepdims=True)
    p = jnp.exp(logits - m)
    att = p / jnp.sum(p, axis=1, keepdims=True)
    out_ref[...] = jnp.sum(e3 * att[:, :, None], axis=1)


@jax.jit
def _tc_mlp(e3, u_rep, w1a, w1b, b1, w2, b2, w3):
    grid = (N_NODES // T,)
    full = lambda shape: pl.BlockSpec(shape, lambda i: (0,) * len(shape))
    return pl.pallas_call(
        _tc_mlp_body,
        grid=grid,
        in_specs=[
            pl.BlockSpec((T, DEG, EMBED), lambda i: (i, 0, 0)),
            pl.BlockSpec((T, EMBED), lambda i: (i, 0)),
            full((EMBED, EMBED)),
            full((EMBED, EMBED)),
            full((1, EMBED)),
            full((EMBED, EMBED)),
            full((1, EMBED)),
            full((1, EMBED)),
        ],
        out_specs=pl.BlockSpec((T, EMBED), lambda i: (i, 0)),
        out_shape=jax.ShapeDtypeStruct((N_NODES, EMBED), jnp.float32),
        compiler_params=pltpu.CompilerParams(
            dimension_semantics=("parallel",)),
    )(e3, u_rep, w1a, w1b, b1, w2, b2, w3)


def kernel(nodes, to_neighs, u2e_weight, W1, b1, W2, b2, W3, b3):
    neigh_flat = to_neighs.reshape(E_ROWS)
    nodes_pad = jnp.concatenate(
        [nodes, jnp.zeros((U_PAD - N_NODES,), jnp.int32)])
    e_gath, u_gath = _sc_gather(neigh_flat, nodes_pad, u2e_weight)
    out = _tc_mlp(
        e_gath.reshape(N_NODES, DEG, EMBED),
        u_gath[:N_NODES],
        W1[:EMBED],
        W1[EMBED:],
        b1.reshape(1, EMBED),
        W2,
        b2.reshape(1, EMBED),
        W3.reshape(1, EMBED),
    )
    return out


# speedup vs baseline: 4.3151x; 1.2975x over previous
import functools

import jax
import jax.numpy as jnp
from jax import lax
from jax.experimental import pallas as pl
from jax.experimental.pallas import tpu as pltpu
from jax.experimental.pallas import tpu_sc as plsc

NUM_USERS = 100000
EMBED = 128
N_NODES = 10000
DEG = 32

NC, NS = 2, 16
NW = NC * NS

E_ROWS = N_NODES * DEG
E_PER_W = E_ROWS // NW
CHUNK = 400
N_CHUNKS = E_PER_W // CHUNK

U_PAD = 10240
U_PER_W = U_PAD // NW


def _sc_gather_body(neigh_hbm, node_hbm, table_hbm,
                    e_out_hbm, u_out_hbm,
                    idx_v, rows_v, idx_u, rows_u, sem):
    wid = lax.axis_index("s") * NC + lax.axis_index("c")
    ebase = wid * E_PER_W

    def chunk(i, _):
        base = ebase + i * CHUNK
        pltpu.sync_copy(neigh_hbm.at[pl.ds(base, CHUNK)], idx_v)
        pltpu.async_copy(table_hbm.at[idx_v], rows_v, sem).wait()
        pltpu.sync_copy(rows_v, e_out_hbm.at[pl.ds(base, CHUNK)])
        return 0

    lax.fori_loop(0, N_CHUNKS, chunk, 0)

    ubase = wid * U_PER_W
    pltpu.sync_copy(node_hbm.at[pl.ds(ubase, U_PER_W)], idx_u)
    pltpu.async_copy(table_hbm.at[idx_u], rows_u, sem).wait()
    pltpu.sync_copy(rows_u, u_out_hbm.at[pl.ds(ubase, U_PER_W)])


@jax.jit
def _sc_gather(neigh_flat, nodes_pad, table):
    mesh = plsc.VectorSubcoreMesh(core_axis_name="c", subcore_axis_name="s")
    return pl.kernel(
        _sc_gather_body,
        out_type=(
            jax.ShapeDtypeStruct((E_ROWS, EMBED), jnp.float32),
            jax.ShapeDtypeStruct((U_PAD, EMBED), jnp.float32),
        ),
        mesh=mesh,
        scratch_types=[
            pltpu.VMEM((CHUNK,), jnp.int32),
            pltpu.VMEM((CHUNK, EMBED), jnp.float32),
            pltpu.VMEM((U_PER_W,), jnp.int32),
            pltpu.VMEM((U_PER_W, EMBED), jnp.float32),
            pltpu.SemaphoreType.DMA,
        ],
    )(neigh_flat, nodes_pad, table)


T = 400


def _tc_mlp_body(e_ref, u_ref, w1a_ref, w1b_ref, b1_ref, w2_ref, b2_ref,
                 w3b_ref, out_ref):
    e3 = e_ref[...]
    e2 = e3.reshape(DEG * T, EMBED)
    u1 = jnp.dot(u_ref[...], w1b_ref[...],
                 preferred_element_type=jnp.float32) + b1_ref[...]
    x = jnp.dot(e2, w1a_ref[...], preferred_element_type=jnp.float32)
    x = jnp.maximum(x.reshape(DEG, T, EMBED) + u1[None, :, :], 0.0)
    x2 = jnp.dot(x.reshape(DEG * T, EMBED), w2_ref[...],
                 preferred_element_type=jnp.float32)
    x2 = jnp.maximum(x2 + b2_ref[...], 0.0)
    p = jnp.exp(jnp.dot(x2, w3b_ref[...],
                        preferred_element_type=jnp.float32))
    p3 = p.reshape(DEG, T, EMBED)
    den = jnp.sum(p3, axis=0)
    num = jnp.sum(p3 * e3, axis=0)
    out_ref[...] = num / den


@jax.jit
def _tc_mlp(e3, u_rep, w1a, w1b, b1, w2, b2, w3b):
    grid = (N_NODES // T,)
    full = lambda shape: pl.BlockSpec(shape, lambda i: (0,) * len(shape))
    return pl.pallas_call(
        _tc_mlp_body,
        grid=grid,
        in_specs=[
            pl.BlockSpec((DEG, T, EMBED), lambda i: (0, i, 0)),
            pl.BlockSpec((T, EMBED), lambda i: (i, 0)),
            full((EMBED, EMBED)),
            full((EMBED, EMBED)),
            full((1, EMBED)),
            full((EMBED, EMBED)),
            full((1, EMBED)),
            full((EMBED, EMBED)),
        ],
        out_specs=pl.BlockSpec((T, EMBED), lambda i: (i, 0)),
        out_shape=jax.ShapeDtypeStruct((N_NODES, EMBED), jnp.float32),
        compiler_params=pltpu.CompilerParams(
            dimension_semantics=("parallel",)),
    )(e3, u_rep, w1a, w1b, b1, w2, b2, w3b)


def kernel(nodes, to_neighs, u2e_weight, W1, b1, W2, b2, W3, b3):
    neigh_flat = to_neighs.T.reshape(E_ROWS)
    nodes_pad = jnp.concatenate(
        [nodes, jnp.zeros((U_PAD - N_NODES,), jnp.int32)])
    e_gath, u_gath = _sc_gather(neigh_flat, nodes_pad, u2e_weight)
    out = _tc_mlp(
        e_gath.reshape(DEG, N_NODES, EMBED),
        u_gath[:N_NODES],
        W1[:EMBED],
        W1[EMBED:],
        b1.reshape(1, EMBED),
        W2,
        b2.reshape(1, EMBED),
        jnp.broadcast_to(W3, (EMBED, EMBED)),
    )
    return out


# speedup vs baseline: 4.6984x; 1.0888x over previous
import functools

import jax
import jax.numpy as jnp
from jax import lax
from jax.experimental import pallas as pl
from jax.experimental.pallas import tpu as pltpu
from jax.experimental.pallas import tpu_sc as plsc

NUM_USERS = 100000
EMBED = 128
N_NODES = 10000
DEG = 32

NC, NS = 2, 16
NW = NC * NS

E_ROWS = N_NODES * DEG
E_PER_W = E_ROWS // NW
CHUNK = 400
N_CHUNKS = E_PER_W // CHUNK

U_PAD = 10240
U_PER_W = U_PAD // NW


def _sc_gather_body(neigh_hbm, node_hbm, table_hbm,
                    e_out_hbm, u_out_hbm,
                    idx_all, rows2, gsem, wsem):
    wid = lax.axis_index("s") * NC + lax.axis_index("c")
    ebase = wid * E_PER_W

    pltpu.sync_copy(neigh_hbm.at[pl.ds(ebase, E_PER_W)], idx_all)

    def start_gather(i, b):
        pltpu.async_copy(
            table_hbm.at[idx_all.at[pl.ds(i * CHUNK, CHUNK)]],
            rows2.at[b], gsem.at[b])

    def gather_wait(b):
        pltpu.make_async_copy(
            table_hbm.at[idx_all.at[pl.ds(0, CHUNK)]],
            rows2.at[b], gsem.at[b]).wait()

    def start_write(i, b):
        pltpu.async_copy(rows2.at[b],
                         e_out_hbm.at[pl.ds(ebase + i * CHUNK, CHUNK)],
                         wsem.at[b])

    def write_wait(b):
        pltpu.make_async_copy(
            rows2.at[b], e_out_hbm.at[pl.ds(ebase, CHUNK)],
            wsem.at[b]).wait()

    start_gather(0, 0)

    def chunk(i, _):
        b = lax.rem(i, 2)
        nb = 1 - b
        gather_wait(b)
        start_write(i, b)

        @pl.when(i + 1 < N_CHUNKS)
        def _():
            @pl.when(i > 0)
            def _():
                write_wait(nb)
            start_gather(i + 1, nb)

        return 0

    lax.fori_loop(0, N_CHUNKS, chunk, 0)
    write_wait(0)
    write_wait(1)

    ubase = wid * U_PER_W
    idx_u = idx_all.at[pl.ds(0, U_PER_W)]
    rows_u = rows2.at[0].at[pl.ds(0, U_PER_W)]
    pltpu.sync_copy(node_hbm.at[pl.ds(ubase, U_PER_W)], idx_u)
    pltpu.async_copy(table_hbm.at[idx_u], rows_u, gsem.at[0]).wait()
    pltpu.sync_copy(rows_u, u_out_hbm.at[pl.ds(ubase, U_PER_W)])


@jax.jit
def _sc_gather(neigh_flat, nodes_pad, table):
    mesh = plsc.VectorSubcoreMesh(core_axis_name="c", subcore_axis_name="s")
    return pl.kernel(
        _sc_gather_body,
        out_type=(
            jax.ShapeDtypeStruct((E_ROWS, EMBED), jnp.float32),
            jax.ShapeDtypeStruct((U_PAD, EMBED), jnp.float32),
        ),
        mesh=mesh,
        scratch_types=[
            pltpu.VMEM((E_PER_W,), jnp.int32),
            pltpu.VMEM((2, CHUNK, EMBED), jnp.float32),
            pltpu.SemaphoreType.DMA((2,)),
            pltpu.SemaphoreType.DMA((2,)),
        ],
    )(neigh_flat, nodes_pad, table)


T = 400


def _tc_mlp_body(e_ref, u_ref, w1a_ref, w1b_ref, b1_ref, w2_ref, b2_ref,
                 w3b_ref, out_ref):
    e3 = e_ref[...]
    e2 = e3.reshape(DEG * T, EMBED)
    u1 = jnp.dot(u_ref[...], w1b_ref[...],
                 preferred_element_type=jnp.float32) + b1_ref[...]
    x = jnp.dot(e2, w1a_ref[...], preferred_element_type=jnp.float32)
    x = jnp.maximum(x.reshape(DEG, T, EMBED) + u1[None, :, :], 0.0)
    x2 = jnp.dot(x.reshape(DEG * T, EMBED), w2_ref[...],
                 preferred_element_type=jnp.float32)
    x2 = jnp.maximum(x2 + b2_ref[...], 0.0)
    p = jnp.exp(jnp.dot(x2, w3b_ref[...],
                        preferred_element_type=jnp.float32))
    p3 = p.reshape(DEG, T, EMBED)
    den = jnp.sum(p3, axis=0)
    num = jnp.sum(p3 * e3, axis=0)
    out_ref[...] = num / den


@jax.jit
def _tc_mlp(e3, u_rep, w1a, w1b, b1, w2, b2, w3b):
    grid = (N_NODES // T,)
    full = lambda shape: pl.BlockSpec(shape, lambda i: (0,) * len(shape))
    return pl.pallas_call(
        _tc_mlp_body,
        grid=grid,
        in_specs=[
            pl.BlockSpec((DEG, T, EMBED), lambda i: (0, i, 0)),
            pl.BlockSpec((T, EMBED), lambda i: (i, 0)),
            full((EMBED, EMBED)),
            full((EMBED, EMBED)),
            full((1, EMBED)),
            full((EMBED, EMBED)),
            full((1, EMBED)),
            full((EMBED, EMBED)),
        ],
        out_specs=pl.BlockSpec((T, EMBED), lambda i: (i, 0)),
        out_shape=jax.ShapeDtypeStruct((N_NODES, EMBED), jnp.float32),
        compiler_params=pltpu.CompilerParams(
            dimension_semantics=("parallel",)),
    )(e3, u_rep, w1a, w1b, b1, w2, b2, w3b)


def kernel(nodes, to_neighs, u2e_weight, W1, b1, W2, b2, W3, b3):
    neigh_flat = to_neighs.T.reshape(E_ROWS)
    nodes_pad = jnp.concatenate(
        [nodes, jnp.zeros((U_PAD - N_NODES,), jnp.int32)])
    e_gath, u_gath = _sc_gather(neigh_flat, nodes_pad, u2e_weight)
    out = _tc_mlp(
        e_gath.reshape(DEG, N_NODES, EMBED),
        u_gath[:N_NODES],
        W1[:EMBED],
        W1[EMBED:],
        b1.reshape(1, EMBED),
        W2,
        b2.reshape(1, EMBED),
        jnp.broadcast_to(W3, (EMBED, EMBED)),
    )
    return out


# speedup vs baseline: 4.8234x; 1.0266x over previous
import functools

import jax
import jax.numpy as jnp
from jax import lax
from jax.experimental import pallas as pl
from jax.experimental.pallas import tpu as pltpu
from jax.experimental.pallas import tpu_sc as plsc

NUM_USERS = 100000
EMBED = 128
N_NODES = 10000
DEG = 32

NC, NS = 2, 16
NW = NC * NS

E_ROWS = N_NODES * DEG
E_PER_W = E_ROWS // NW
CHUNK = 400
N_CHUNKS = E_PER_W // CHUNK

U_PAD = 10240
U_PER_W = U_PAD // NW


def _sc_gather_body(neigh_hbm, node_hbm, table_hbm,
                    e_out_hbm, u_out_hbm,
                    idx_all, rows2, gsem, wsem):
    wid = lax.axis_index("s") * NC + lax.axis_index("c")
    ebase = wid * E_PER_W

    pltpu.sync_copy(neigh_hbm.at[pl.ds(ebase, E_PER_W)], idx_all)

    def start_gather(i, b):
        pltpu.async_copy(
            table_hbm.at[idx_all.at[pl.ds(i * CHUNK, CHUNK)]],
            rows2.at[b], gsem.at[b])

    def gather_wait(b):
        pltpu.make_async_copy(
            table_hbm.at[idx_all.at[pl.ds(0, CHUNK)]],
            rows2.at[b], gsem.at[b]).wait()

    def start_write(i, b):
        pltpu.async_copy(rows2.at[b],
                         e_out_hbm.at[pl.ds(ebase + i * CHUNK, CHUNK)],
                         wsem.at[b])

    def write_wait(b):
        pltpu.make_async_copy(
            rows2.at[b], e_out_hbm.at[pl.ds(ebase, CHUNK)],
            wsem.at[b]).wait()

    start_gather(0, 0)

    def chunk(i, _):
        b = lax.rem(i, 2)
        nb = 1 - b
        gather_wait(b)
        start_write(i, b)

        @pl.when(i + 1 < N_CHUNKS)
        def _():
            @pl.when(i > 0)
            def _():
                write_wait(nb)
            start_gather(i + 1, nb)

        return 0

    lax.fori_loop(0, N_CHUNKS, chunk, 0)
    write_wait(0)
    write_wait(1)

    ubase = wid * U_PER_W
    idx_u = idx_all.at[pl.ds(0, U_PER_W)]
    rows_u = rows2.at[0].at[pl.ds(0, U_PER_W)]
    pltpu.sync_copy(node_hbm.at[pl.ds(ubase, U_PER_W)], idx_u)
    pltpu.async_copy(table_hbm.at[idx_u], rows_u, gsem.at[0]).wait()
    pltpu.sync_copy(rows_u, u_out_hbm.at[pl.ds(ubase, U_PER_W)])


@jax.jit
def _sc_gather(neigh_flat, nodes_pad, table):
    mesh = plsc.VectorSubcoreMesh(core_axis_name="c", subcore_axis_name="s")
    return pl.kernel(
        _sc_gather_body,
        out_type=(
            jax.ShapeDtypeStruct((E_ROWS, EMBED), jnp.float32),
            jax.ShapeDtypeStruct((U_PAD, EMBED), jnp.float32),
        ),
        mesh=mesh,
        scratch_types=[
            pltpu.VMEM((E_PER_W,), jnp.int32),
            pltpu.VMEM((2, CHUNK, EMBED), jnp.float32),
            pltpu.SemaphoreType.DMA((2,)),
            pltpu.SemaphoreType.DMA((2,)),
        ],
    )(neigh_flat, nodes_pad, table)


T = 400


def _tc_mlp_body(e_ref, u_ref, w1a_ref, w1b_ref, b1_ref, w2_ref, b2_ref,
                 w3b_ref, out_ref):
    e3 = e_ref[...]
    e2 = e3.reshape(DEG * T, EMBED).astype(jnp.bfloat16)
    u1 = jnp.dot(u_ref[...], w1b_ref[...],
                 preferred_element_type=jnp.float32) + b1_ref[...]
    x = jnp.dot(e2, w1a_ref[...], preferred_element_type=jnp.float32)
    x = jnp.maximum(x.reshape(DEG, T, EMBED) + u1[None, :, :], 0.0)
    xb = x.reshape(DEG * T, EMBED).astype(jnp.bfloat16)
    x2 = jnp.dot(xb, w2_ref[...], preferred_element_type=jnp.float32)
    x2 = jnp.maximum(x2 + b2_ref[...], 0.0).astype(jnp.bfloat16)
    p = jnp.exp(jnp.dot(x2, w3b_ref[...],
                        preferred_element_type=jnp.float32))
    p3 = p.reshape(DEG, T, EMBED)
    den = jnp.sum(p3, axis=0)
    num = jnp.sum(p3 * e3, axis=0)
    out_ref[...] = num / den


@jax.jit
def _tc_mlp(e3, u_rep, w1a, w1b, b1, w2, b2, w3b):
    grid = (N_NODES // T,)
    full = lambda shape: pl.BlockSpec(shape, lambda i: (0,) * len(shape))
    return pl.pallas_call(
        _tc_mlp_body,
        grid=grid,
        in_specs=[
            pl.BlockSpec((DEG, T, EMBED), lambda i: (0, i, 0)),
            pl.BlockSpec((T, EMBED), lambda i: (i, 0)),
            full((EMBED, EMBED)),
            full((EMBED, EMBED)),
            full((1, EMBED)),
            full((EMBED, EMBED)),
            full((1, EMBED)),
            full((EMBED, EMBED)),
        ],
        out_specs=pl.BlockSpec((T, EMBED), lambda i: (i, 0)),
        out_shape=jax.ShapeDtypeStruct((N_NODES, EMBED), jnp.float32),
        compiler_params=pltpu.CompilerParams(
            dimension_semantics=("parallel",)),
    )(e3, u_rep, w1a, w1b, b1, w2, b2, w3b)


def kernel(nodes, to_neighs, u2e_weight, W1, b1, W2, b2, W3, b3):
    neigh_flat = to_neighs.T.reshape(E_ROWS)
    nodes_pad = jnp.concatenate(
        [nodes, jnp.zeros((U_PAD - N_NODES,), jnp.int32)])
    e_gath, u_gath = _sc_gather(neigh_flat, nodes_pad, u2e_weight)
    out = _tc_mlp(
        e_gath.reshape(DEG, N_NODES, EMBED),
        u_gath[:N_NODES],
        W1[:EMBED].astype(jnp.bfloat16),
        W1[EMBED:],
        b1.reshape(1, EMBED),
        W2.astype(jnp.bfloat16),
        b2.reshape(1, EMBED),
        jnp.broadcast_to(W3, (EMBED, EMBED)).astype(jnp.bfloat16),
    )
    return out
